# baseline (speedup 1.0000x reference)
import functools

import jax
import jax.numpy as jnp
from jax import lax
from jax.experimental import pallas as pl
from jax.experimental.pallas import tpu as pltpu
from jax.experimental.pallas import tpu_sc as plsc

NC = 2
NS = 16
NW = NC * NS
CH = 128
L = 16
SPLIT0 = 1.9


@functools.lru_cache(maxsize=None)
def _build(N, E, D, H):
    NCHW = ((-(-E // (NW * CH)) + 7) // 8) * 8
    EP = NW * CH * NCHW
    NP = ((N + 1 + 127) // 128) * 128
    RPT = NP // NS
    BR = NP // 8

    mesh_deg = plsc.VectorSubcoreMesh(core_axis_name="c", subcore_axis_name="s")
    mesh_gs = plsc.VectorSubcoreMesh(core_axis_name="c", subcore_axis_name="s")

    EPW = NCHW * CH

    @functools.partial(
        pl.kernel,
        out_type=jax.ShapeDtypeStruct((NW * NP,), jnp.float32),
        mesh=mesh_deg,
        scratch_types=[
            pltpu.VMEM((EPW,), jnp.int32),
            pltpu.VMEM((NP,), jnp.float32),
        ],
        compiler_params=pltpu.CompilerParams(needs_layout_passes=False),
    )
    def sc_deg(dst_hbm, zer_hbm, out_hbm, didx, hist):
        c = lax.axis_index("c")
        s = lax.axis_index("s")
        wid = s * NC + c
        pltpu.sync_copy(zer_hbm, hist)
        pltpu.sync_copy(dst_hbm.at[pl.ds(wid * EPW, EPW)], didx)
        one = jnp.full((L,), 1.0, jnp.float32)

        def body(k, carry):
            iv = didx[pl.ds(k * L, L)]
            plsc.addupdate_scatter(hist, [iv], one)
            return carry

        lax.fori_loop(0, EPW // L, body, 0)
        pltpu.sync_copy(hist, out_hbm.at[pl.ds(wid * NP, NP)])

    A = 2 * NCHW
    assert A % 8 == 0
    G = 24
    ZB = 8

    @functools.partial(
        pl.kernel,
        out_type=jax.ShapeDtypeStruct((NP, H), jnp.float32),
        mesh=mesh_gs,
        scratch_types=[
            pltpu.VMEM((2 * G, CH), jnp.int32),
            pltpu.VMEM((2 * G, CH), jnp.int32),
            pltpu.VMEM((2 * CH, H), jnp.float32),
            pltpu.VMEM((ZB, H), jnp.float32),
            pltpu.VMEM_SHARED((NP, H), jnp.float32),
            pltpu.SemaphoreType.DMA,
        ],
    )
    def sc_gs(y_hbm, src_hbm, dst_hbm, out_hbm,
              sidx, didx, rows, zbuf, acc, sem):
        c = lax.axis_index("c")
        s = lax.axis_index("s")

        @pl.when(c == 0)
        def _core0():
            _core0_body(y_hbm, src_hbm, dst_hbm, out_hbm,
                        sidx, didx, rows, zbuf, acc, sem, s)

    def _core0_body(y_hbm, src_hbm, dst_hbm, out_hbm,
                    sidx, didx, rows, zbuf, acc, sem, s):
        zv = jnp.zeros((L,), jnp.float32)
        for zr in range(ZB):
            for zc in range(H // L):
                zbuf[zr, pl.ds(zc * L, L)] = zv

        def zbody(r, carry):
            @pl.when(r < RPT // ZB)
            def _():
                pltpu.async_copy(zbuf,
                                 acc.at[pl.ds(s * RPT + r * ZB, ZB)], sem)

            @pl.when(r >= 8)
            def _():
                pltpu.make_async_copy(
                    zbuf, acc.at[pl.ds(s * RPT, ZB)], sem).wait()
            return carry

        lax.fori_loop(0, RPT // ZB + 8, zbody, 0)

        base = s * A
        pltpu.sync_copy(src_hbm.at[pl.ds(base, G)], sidx.at[pl.ds(0, G)])
        pltpu.sync_copy(dst_hbm.at[pl.ds(base, G)], didx.at[pl.ds(0, G)])

        plsc.subcore_barrier()
        pltpu.async_copy(y_hbm.at[sidx.at[0]], rows.at[pl.ds(0, CH)], sem)

        def body(j, carry):
            b = lax.rem(j, 2) * CH
            g2 = lax.rem(j + 1, 2 * G)
            pltpu.make_async_copy(y_hbm.at[pl.ds(0, CH)],
                                  rows.at[pl.ds(0, CH)], sem).wait()

            @pl.when(lax.rem(j + 1, G) == 0)
            def _():
                off = pl.multiple_of(base + j + 1, 8)
                goff = pl.multiple_of(g2, 8)
                pltpu.sync_copy(src_hbm.at[pl.ds(off, G)],
                                sidx.at[pl.ds(goff, G)])
                pltpu.sync_copy(dst_hbm.at[pl.ds(off, G)],
                                didx.at[pl.ds(goff, G)])

            pltpu.async_copy(y_hbm.at[sidx.at[g2]],
                             rows.at[pl.ds(CH - b, CH)], sem)
            pltpu.sync_copy(rows.at[pl.ds(b, CH)],
                            acc.at[didx.at[lax.rem(j, 2 * G)]], add=True)
            return carry

        lax.fori_loop(0, A, body, 0)
        pltpu.make_async_copy(y_hbm.at[pl.ds(0, CH)],
                              rows.at[pl.ds(0, CH)], sem).wait()
        plsc.subcore_barrier()
        pltpu.sync_copy(acc.at[pl.ds(s * RPT, RPT)],
                        out_hbm.at[pl.ds(s * RPT, RPT)])

    def dinv_of(pt_ref):
        deg = jnp.sum(pt_ref[...], axis=1, keepdims=True) + 1.0
        return lax.rsqrt(deg)

    def tc1_body(pt, x, w, y):
        dinv = dinv_of(pt)
        y[...] = jnp.dot(x[...], w[...],
                         preferred_element_type=jnp.float32) * dinv

    grid = NP // BR
    rowblk = lambda wdt: pl.BlockSpec((BR, wdt), lambda i: (i, 0))
    full = lambda a, b: pl.BlockSpec((a, b), lambda i: (0, 0))

    tc1 = pl.pallas_call(
        tc1_body,
        grid=(grid,),
        in_specs=[rowblk(NW), rowblk(D), full(D, H)],
        out_specs=rowblk(H),
        out_shape=jax.ShapeDtypeStruct((NP, H), jnp.float32),
    )

    def tc2_body(q0, y1, pt, b, w, o):
        i = pl.program_id(0)
        dinv = dinv_of(pt)
        h = jnp.maximum((q0[...] + y1[...]) * dinv + b[...], 0.0)
        y2 = jnp.dot(h, w[...], preferred_element_type=jnp.float32) * dinv
        row = i * BR + lax.broadcasted_iota(jnp.int32, (BR, 1), 0)
        o[...] = jnp.where(row < N, y2, 0.0)

    tc2 = pl.pallas_call(
        tc2_body,
        grid=(grid,),
        in_specs=[rowblk(H), rowblk(H), rowblk(NW),
                  full(1, H), full(H, H)],
        out_specs=rowblk(H),
        out_shape=jax.ShapeDtypeStruct((NP, H), jnp.float32),
    )

    def tc3_body(r0, y2, pt, b, o):
        dinv = dinv_of(pt)
        o[...] = jnp.maximum((r0[...] + y2[...]) * dinv + b[...], 0.0)

    tc3 = pl.pallas_call(
        tc3_body,
        grid=(grid,),
        in_specs=[rowblk(H), rowblk(H), rowblk(NW),
                  full(1, H)],
        out_specs=rowblk(H),
        out_shape=jax.ShapeDtypeStruct((N, H), jnp.float32),
    )

    return NCHW, EP + G * CH, NP, RPT, sc_deg, sc_gs, tc1, tc2, tc3


def kernel(x, edge_index, W1, b1, W2, b2):
    N, D = x.shape
    H = W1.shape[1]
    E = edge_index.shape[1]
    NCHW, EP, NP, RPT, sc_deg, sc_gs, tc1, tc2, tc3 = _build(N, E, D, H)

    padv = jnp.full((EP - E,), N, jnp.int32)
    dst1d = jnp.concatenate([edge_index[1], padv])
    srcp = jnp.concatenate([edge_index[0], padv]).reshape(-1, CH)
    dstp = dst1d.reshape(-1, CH)
    xp = jnp.concatenate([x, jnp.zeros((NP - N, D), x.dtype)])
    zdeg = jnp.zeros((NP,), jnp.float32)

    degp = sc_deg(dst1d, zdeg)
    pt = degp.reshape(NW, NP).T
    y1 = tc1(pt, xp, W1)
    q = sc_gs(y1, srcp, dstp)
    y2 = tc2(q, y1, pt, b1.reshape(1, H), W2)
    r = sc_gs(y2, srcp, dstp)
    return tc3(r, y2, pt, b2.reshape(1, H))


# speedup vs baseline: 1.3196x; 1.3196x over previous
import functools

import jax
import jax.numpy as jnp
from jax import lax
from jax.experimental import pallas as pl
from jax.experimental.pallas import tpu as pltpu
from jax.experimental.pallas import tpu_sc as plsc

NC = 2
NS = 16
NW = NC * NS
CH = 128
L = 16
SPLIT0 = 1.9


@functools.lru_cache(maxsize=None)
def _build(N, E, D, H):
    NCHW = ((-(-E // (NW * CH)) + 7) // 8) * 8
    EP = NW * CH * NCHW
    NP = ((N + 1 + 127) // 128) * 128
    RPT = NP // NS
    BR = NP // 8

    mesh_deg = plsc.VectorSubcoreMesh(core_axis_name="c", subcore_axis_name="s")
    mesh_gs = plsc.VectorSubcoreMesh(core_axis_name="c", subcore_axis_name="s")

    EPW = NCHW * CH

    @functools.partial(
        pl.kernel,
        out_type=jax.ShapeDtypeStruct((NW * NP,), jnp.float32),
        mesh=mesh_deg,
        scratch_types=[
            pltpu.VMEM((EPW,), jnp.int32),
            pltpu.VMEM((NP,), jnp.float32),
        ],
        compiler_params=pltpu.CompilerParams(needs_layout_passes=False),
    )
    def sc_deg(dst_hbm, zer_hbm, out_hbm, didx, hist):
        c = lax.axis_index("c")
        s = lax.axis_index("s")
        wid = s * NC + c
        pltpu.sync_copy(zer_hbm, hist)
        pltpu.sync_copy(dst_hbm.at[pl.ds(wid * EPW, EPW)], didx)
        one = jnp.full((L,), 1.0, jnp.float32)

        def body(k, carry):
            iv = didx[pl.ds(k * L, L)]
            plsc.addupdate_scatter(hist, [iv], one)
            return carry

        lax.fori_loop(0, EPW // L, body, 0)
        pltpu.sync_copy(hist, out_hbm.at[pl.ds(wid * NP, NP)])

    A = int(NCHW * SPLIT0) // 8 * 8
    B = 2 * NCHW - A
    assert A % 8 == 0 and B % 8 == 0 and A >= 4 and B >= 4
    G = 24
    ZB = 8

    @functools.partial(
        pl.kernel,
        out_type=jax.ShapeDtypeStruct((NC * NP, H), jnp.float32),
        mesh=mesh_gs,
        scratch_types=[
            pltpu.VMEM((2 * G, CH), jnp.int32),
            pltpu.VMEM((2 * G, CH), jnp.int32),
            pltpu.VMEM((2 * CH, H), jnp.float32),
            pltpu.VMEM((ZB, H), jnp.float32),
            pltpu.VMEM_SHARED((NP, H), jnp.float32),
            pltpu.SemaphoreType.DMA,
        ],
    )
    def sc_gs(y_hbm, src_hbm, dst_hbm, out_hbm,
              sidx, didx, rows, zbuf, acc, sem):
        c = lax.axis_index("c")
        s = lax.axis_index("s")

        zv = jnp.zeros((L,), jnp.float32)
        for zr in range(ZB):
            for zc in range(H // L):
                zbuf[zr, pl.ds(zc * L, L)] = zv

        def zbody(r, carry):
            @pl.when(r < RPT // ZB)
            def _():
                pltpu.async_copy(zbuf,
                                 acc.at[pl.ds(s * RPT + r * ZB, ZB)], sem)

            @pl.when(r >= 8)
            def _():
                pltpu.make_async_copy(
                    zbuf, acc.at[pl.ds(s * RPT, ZB)], sem).wait()
            return carry

        lax.fori_loop(0, RPT // ZB + 8, zbody, 0)

        base = jnp.where(c == 0, s * A, 16 * A + s * B)
        nch = jnp.where(c == 0, A, B)
        pltpu.sync_copy(src_hbm.at[pl.ds(base, G)], sidx.at[pl.ds(0, G)])
        pltpu.sync_copy(dst_hbm.at[pl.ds(base, G)], didx.at[pl.ds(0, G)])

        plsc.subcore_barrier()
        pltpu.async_copy(y_hbm.at[sidx.at[0]], rows.at[pl.ds(0, CH)], sem)

        def body(j, carry):
            b = lax.rem(j, 2) * CH
            g2 = lax.rem(j + 1, 2 * G)
            pltpu.make_async_copy(y_hbm.at[pl.ds(0, CH)],
                                  rows.at[pl.ds(0, CH)], sem).wait()

            @pl.when(lax.rem(j + 1, G) == 0)
            def _():
                off = pl.multiple_of(base + j + 1, 8)
                goff = pl.multiple_of(g2, 8)
                pltpu.sync_copy(src_hbm.at[pl.ds(off, G)],
                                sidx.at[pl.ds(goff, G)])
                pltpu.sync_copy(dst_hbm.at[pl.ds(off, G)],
                                didx.at[pl.ds(goff, G)])

            pltpu.async_copy(y_hbm.at[sidx.at[g2]],
                             rows.at[pl.ds(CH - b, CH)], sem)
            pltpu.sync_copy(rows.at[pl.ds(b, CH)],
                            acc.at[didx.at[lax.rem(j, 2 * G)]], add=True)
            return carry

        lax.fori_loop(0, nch, body, 0)
        pltpu.make_async_copy(y_hbm.at[pl.ds(0, CH)],
                              rows.at[pl.ds(0, CH)], sem).wait()
        plsc.subcore_barrier()
        pltpu.sync_copy(acc.at[pl.ds(s * RPT, RPT)],
                        out_hbm.at[pl.ds(c * NP + s * RPT, RPT)])

    def dinv_of(pt_ref):
        deg = jnp.sum(pt_ref[...], axis=1, keepdims=True) + 1.0
        return lax.rsqrt(deg)

    def tc1_body(pt, x, w, y):
        dinv = dinv_of(pt)
        y[...] = jnp.dot(x[...], w[...],
                         preferred_element_type=jnp.float32) * dinv

    grid = NP // BR
    rowblk = lambda wdt: pl.BlockSpec((BR, wdt), lambda i: (i, 0))
    full = lambda a, b: pl.BlockSpec((a, b), lambda i: (0, 0))

    tc1 = pl.pallas_call(
        tc1_body,
        grid=(grid,),
        in_specs=[rowblk(NW), rowblk(D), full(D, H)],
        out_specs=rowblk(H),
        out_shape=jax.ShapeDtypeStruct((NP, H), jnp.float32),
    )

    def tc2_body(q0, q1, y1, pt, b, w, o):
        i = pl.program_id(0)
        dinv = dinv_of(pt)
        h = jnp.maximum((q0[...] + q1[...] + y1[...]) * dinv + b[...], 0.0)
        y2 = jnp.dot(h, w[...], preferred_element_type=jnp.float32) * dinv
        row = i * BR + lax.broadcasted_iota(jnp.int32, (BR, 1), 0)
        o[...] = jnp.where(row < N, y2, 0.0)

    tc2 = pl.pallas_call(
        tc2_body,
        grid=(grid,),
        in_specs=[rowblk(H), rowblk(H), rowblk(H), rowblk(NW),
                  full(1, H), full(H, H)],
        out_specs=rowblk(H),
        out_shape=jax.ShapeDtypeStruct((NP, H), jnp.float32),
    )

    def tc3_body(r0, r1, y2, pt, b, o):
        dinv = dinv_of(pt)
        o[...] = jnp.maximum((r0[...] + r1[...] + y2[...]) * dinv + b[...], 0.0)

    tc3 = pl.pallas_call(
        tc3_body,
        grid=(grid,),
        in_specs=[rowblk(H), rowblk(H), rowblk(H), rowblk(NW),
                  full(1, H)],
        out_specs=rowblk(H),
        out_shape=jax.ShapeDtypeStruct((N, H), jnp.float32),
    )

    return NCHW, EP + G * CH, NP, RPT, sc_deg, sc_gs, tc1, tc2, tc3


def kernel(x, edge_index, W1, b1, W2, b2):
    N, D = x.shape
    H = W1.shape[1]
    E = edge_index.shape[1]
    NCHW, EP, NP, RPT, sc_deg, sc_gs, tc1, tc2, tc3 = _build(N, E, D, H)

    padv = jnp.full((EP - E,), N, jnp.int32)
    dst1d = jnp.concatenate([edge_index[1], padv])
    srcp = jnp.concatenate([edge_index[0], padv]).reshape(-1, CH)
    dstp = dst1d.reshape(-1, CH)
    xp = jnp.concatenate([x, jnp.zeros((NP - N, D), x.dtype)])
    zdeg = jnp.zeros((NP,), jnp.float32)

    degp = sc_deg(dst1d, zdeg)
    pt = degp.reshape(NW, NP).T
    y1 = tc1(pt, xp, W1)
    q = sc_gs(y1, srcp, dstp)
    y2 = tc2(q[:NP], q[NP:], y1, pt, b1.reshape(1, H), W2)
    r = sc_gs(y2, srcp, dstp)
    return tc3(r[:NP], r[NP:], y2, pt, b2.reshape(1, H))
